# Initial kernel scaffold; baseline (speedup 1.0000x reference)
import jax
import jax.numpy as jnp
from jax.experimental import pallas as pl


def kernel(node_states, from_idx, to_idx, edge_features, graph_idx, mW1, mb1, mW2, mb2, mW3, mb3, g0_Wih, g0_Whh, g0_bih, g0_bhh, g1_Wih, g1_Whh, g1_bih, g1_bhh, g2_Wih, g2_Whh, g2_bih, g2_bhh):
    raise NotImplementedError("write your pallas kernel here")


# speedup vs baseline: 2.3744x; 2.3744x over previous
import functools

import jax
import jax.numpy as jnp
from jax import lax
from jax.experimental import pallas as pl
from jax.experimental.pallas import tpu as pltpu
from jax.experimental.pallas import tpu_sc as plsc

_NW = 32
_BE = 80
_HN = 5120
_NA = 5632


def _proj_body(ns_ref, w_ref, s_ref):
    s_ref[...] = jnp.dot(ns_ref[...], w_ref[...],
                         preferred_element_type=jnp.float32)


def _node_proj(ns, w_ab):
    n, d = ns.shape
    bn = 2000
    return pl.pallas_call(
        _proj_body,
        grid=(n // bn,),
        in_specs=[pl.BlockSpec((bn, d), lambda i: (i, 0)),
                  pl.BlockSpec(w_ab.shape, lambda i: (0, 0))],
        out_specs=pl.BlockSpec((bn, w_ab.shape[1]), lambda i: (i, 0)),
        out_shape=jax.ShapeDtypeStruct((n, w_ab.shape[1]), jnp.float32),
    )(ns, w_ab)


def _sc_gather(s_tab, fi3, ti3):
    nblk = fi3.shape[1]
    e = _NW * nblk * _BE
    dd = s_tab.shape[1]
    mesh = plsc.VectorSubcoreMesh(core_axis_name="c", subcore_axis_name="s")

    @functools.partial(
        pl.kernel, mesh=mesh,
        out_type=[jax.ShapeDtypeStruct((e, dd), jnp.float32),
                  jax.ShapeDtypeStruct((e, dd), jnp.float32)],
        scratch_types=[
            pltpu.VMEM((nblk, _BE), jnp.int32),
            pltpu.VMEM((nblk, _BE), jnp.int32),
            pltpu.VMEM((_BE, dd), jnp.float32),
            pltpu.VMEM((_BE, dd), jnp.float32),
            pltpu.SemaphoreType.DMA,
            pltpu.SemaphoreType.DMA,
            pltpu.SemaphoreType.DMA,
            pltpu.SemaphoreType.DMA,
        ],
    )
    def k(s_hbm, fi_hbm, ti_hbm, sf_hbm, st_hbm,
          fiv, tiv, bf, bt, sg0, sg1, so0, so1):
        cid = lax.axis_index("c")
        sid = lax.axis_index("s")
        wid = sid * 2 + cid
        base = wid * (nblk * _BE)
        pltpu.sync_copy(fi_hbm.at[wid], fiv)
        pltpu.sync_copy(ti_hbm.at[wid], tiv)

        def body(j, carry):
            r0 = base + j * _BE
            cg0 = pltpu.async_copy(s_hbm.at[fiv.at[j]], bf, sg0)
            cg1 = pltpu.async_copy(s_hbm.at[tiv.at[j]], bt, sg1)
            cg0.wait()
            cg1.wait()
            co0 = pltpu.async_copy(bf, sf_hbm.at[pl.ds(r0, _BE)], so0)
            co1 = pltpu.async_copy(bt, st_hbm.at[pl.ds(r0, _BE)], so1)
            co0.wait()
            co1.wait()
            return carry

        lax.fori_loop(0, nblk, body, 0)

    return k(s_tab, fi3, ti3)


def _tail_body(ef_ref, sf_ref, st_ref, w1c_ref, b1_ref, w2_ref, b2_ref,
               w3_ref, b3_ref, e1_ref, e2_ref):
    d = ef_ref.shape[1]
    c = jnp.dot(ef_ref[...], w1c_ref[...],
                preferred_element_type=jnp.float32) + b1_ref[...]
    sf = sf_ref[...]
    st = st_ref[...]

    def head(g):
        h1 = jnp.maximum(g, 0.0)
        h2 = jnp.dot(h1, w2_ref[...], preferred_element_type=jnp.float32)
        h2 = jnp.maximum(h2 + b2_ref[...], 0.0)
        return jnp.dot(h2, w3_ref[...],
                       preferred_element_type=jnp.float32) + b3_ref[...]

    e1_ref[...] = head(sf[:, :d] + st[:, d:] + c)
    e2_ref[...] = head(st[:, :d] + sf[:, d:] + c)


def _mlp_tail(ef, sf, st, w1c, b1, w2, b2, w3, b3):
    e, d = ef.shape
    h = w1c.shape[1]
    be = 2000
    wspec = lambda shape: pl.BlockSpec(shape, lambda i: (0, 0))
    return pl.pallas_call(
        _tail_body,
        grid=(e // be,),
        in_specs=[pl.BlockSpec((be, d), lambda i: (i, 0)),
                  pl.BlockSpec((be, 2 * d), lambda i: (i, 0)),
                  pl.BlockSpec((be, 2 * d), lambda i: (i, 0)),
                  wspec(w1c.shape), wspec(b1.shape),
                  wspec(w2.shape), wspec(b2.shape),
                  wspec(w3.shape), wspec(b3.shape)],
        out_specs=[pl.BlockSpec((be, h), lambda i: (i, 0)),
                   pl.BlockSpec((be, h), lambda i: (i, 0))],
        out_shape=[jax.ShapeDtypeStruct((e, h), jnp.float32),
                   jax.ShapeDtypeStruct((e, h), jnp.float32)],
    )(ef, sf, st, w1c, b1, w2, b2, w3, b3)


def _sc_scatter(es1, es2, ti3, fi3, zeros):
    na, d = zeros.shape
    nblk = ti3.shape[1]
    rpt = na // 16
    wpr = _BE // 16
    mesh = plsc.VectorSubcoreMesh(core_axis_name="c", subcore_axis_name="s")

    @functools.partial(
        pl.kernel, mesh=mesh,
        out_type=jax.ShapeDtypeStruct((2 * _HN, d), jnp.float32),
        scratch_types=[
            pltpu.VMEM((nblk, _BE), jnp.int32),
            pltpu.VMEM((nblk, _BE), jnp.int32),
            pltpu.VMEM((_BE, d), jnp.float32),
            pltpu.VMEM((_BE, d), jnp.float32),
            pltpu.VMEM_SHARED((na, d), jnp.float32),
            pltpu.SemaphoreType.DMA,
            pltpu.SemaphoreType.DMA,
        ],
    )
    def k(es1_hbm, es2_hbm, ti_hbm, fi_hbm, z_hbm, out_hbm,
          tiv, fiv, b1, b2, acc, s0, s1):
        cid = lax.axis_index("c")
        sid = lax.axis_index("s")
        base = sid * (nblk * _BE)
        tr0 = sid * rpt
        pltpu.sync_copy(z_hbm.at[pl.ds(tr0, rpt)], acc.at[pl.ds(tr0, rpt)])
        pltpu.sync_copy(ti_hbm.at[sid], tiv)
        pltpu.sync_copy(fi_hbm.at[sid], fiv)

        lo = cid * _HN

        def rebase(j, carry):
            for ref in (tiv, fiv):
                for w in range(wpr):
                    v = ref[j, pl.ds(w * 16, 16)] - lo
                    oob = (v < 0) | (v >= _HN)
                    ref[j, pl.ds(w * 16, 16)] = jnp.where(oob, _HN, v)
            return carry

        lax.fori_loop(0, nblk, rebase, 0)
        plsc.subcore_barrier()

        def body(j, carry):
            r0 = base + j * _BE
            c0 = pltpu.async_copy(es1_hbm.at[pl.ds(r0, _BE)], b1, s0)
            c1 = pltpu.async_copy(es2_hbm.at[pl.ds(r0, _BE)], b2, s1)
            c0.wait()
            c1.wait()
            pltpu.sync_copy(b1, acc.at[tiv.at[j]], add=True)
            pltpu.sync_copy(b2, acc.at[fiv.at[j]], add=True)
            return carry

        lax.fori_loop(0, nblk, body, 0)
        plsc.subcore_barrier()
        drain = _HN // 16
        pltpu.sync_copy(acc.at[pl.ds(sid * drain, drain)],
                        out_hbm.at[pl.ds(cid * _HN + sid * drain, drain)])

    return k(es1, es2, ti3, fi3, zeros)


def _gru_body(ns_ref, agg_ref,
              wih0, whh0, bih0, bhh0,
              wih1, whh1, bih1, bhh1,
              wih2, whh2, bih2, bhh2, out_ref):
    d = ns_ref.shape[1]

    def cell(x, h, wih, whh, bih, bhh):
        gi = jnp.dot(x, wih[...], preferred_element_type=jnp.float32) + bih[...]
        gh = jnp.dot(h, whh[...], preferred_element_type=jnp.float32) + bhh[...]
        r = jax.nn.sigmoid(gi[:, :d] + gh[:, :d])
        z = jax.nn.sigmoid(gi[:, d:2 * d] + gh[:, d:2 * d])
        nn = jnp.tanh(gi[:, 2 * d:] + r * gh[:, 2 * d:])
        return (1.0 - z) * nn + z * h

    x0 = ns_ref[...]
    agg = agg_ref[...]
    n1 = cell(x0, agg, wih0, whh0, bih0, bhh0)
    n2 = cell(agg, n1, wih1, whh1, bih1, bhh1)
    out_ref[...] = cell(n1, n2, wih2, whh2, bih2, bhh2)


def _gru(ns, agg, weights):
    n, d = ns.shape
    bn = 2000
    wspecs = [pl.BlockSpec(w.shape, lambda i: (0, 0)) for w in weights]
    return pl.pallas_call(
        _gru_body,
        grid=(n // bn,),
        in_specs=[pl.BlockSpec((bn, d), lambda i: (i, 0)),
                  pl.BlockSpec((bn, d), lambda i: (i, 0))] + wspecs,
        out_specs=pl.BlockSpec((bn, d), lambda i: (i, 0)),
        out_shape=jax.ShapeDtypeStruct((n, d), jnp.float32),
    )(ns, agg, *weights)


def kernel(node_states, from_idx, to_idx, edge_features, graph_idx,
           mW1, mb1, mW2, mb2, mW3, mb3,
           g0_Wih, g0_Whh, g0_bih, g0_bhh,
           g1_Wih, g1_Whh, g1_bih, g1_bhh,
           g2_Wih, g2_Whh, g2_bih, g2_bhh):
    n, d = node_states.shape
    e = from_idx.shape[0]

    w_ab = jnp.concatenate([mW1[:d], mW1[d:2 * d]], axis=1)
    s_tab = _node_proj(node_states, w_ab)

    fi3 = from_idx.reshape(_NW, -1, _BE)
    ti3 = to_idx.reshape(_NW, -1, _BE)
    sf, st = _sc_gather(s_tab, fi3, ti3)

    es1, es2 = _mlp_tail(edge_features, sf, st,
                         mW1[2 * d:], mb1.reshape(1, -1),
                         mW2, mb2.reshape(1, -1),
                         mW3, mb3.reshape(1, -1))

    ti3s = to_idx.reshape(16, -1, _BE)
    fi3s = from_idx.reshape(16, -1, _BE)
    zeros = jnp.zeros((_NA, d), jnp.float32)
    agg = _sc_scatter(es1, es2, ti3s, fi3s, zeros)

    weights = [g0_Wih.T, g0_Whh.T, g0_bih.reshape(1, -1), g0_bhh.reshape(1, -1),
               g1_Wih.T, g1_Whh.T, g1_bih.reshape(1, -1), g1_bhh.reshape(1, -1),
               g2_Wih.T, g2_Whh.T, g2_bih.reshape(1, -1), g2_bhh.reshape(1, -1)]
    return _gru(node_states, agg[:n], weights)


# speedup vs baseline: 2.7084x; 1.1407x over previous
import functools

import jax
import jax.numpy as jnp
from jax import lax
from jax.experimental import pallas as pl
from jax.experimental.pallas import tpu as pltpu
from jax.experimental.pallas import tpu_sc as plsc

_NW = 32
_BE = 80
_HN = 5120
_NA = 5632


def _proj_body(ns_ref, w_ref, s_ref):
    s_ref[...] = jnp.dot(ns_ref[...], w_ref[...],
                         preferred_element_type=jnp.float32)


def _node_proj(ns, w_ab):
    n, d = ns.shape
    bn = 2000
    return pl.pallas_call(
        _proj_body,
        grid=(n // bn,),
        in_specs=[pl.BlockSpec((bn, d), lambda i: (i, 0)),
                  pl.BlockSpec(w_ab.shape, lambda i: (0, 0))],
        out_specs=pl.BlockSpec((bn, w_ab.shape[1]), lambda i: (i, 0)),
        out_shape=jax.ShapeDtypeStruct((n, w_ab.shape[1]), jnp.float32),
    )(ns, w_ab)


def _sc_gather(s_tab, fi3, ti3):
    nblk = fi3.shape[1]
    e = _NW * nblk * _BE
    dd = s_tab.shape[1]
    mesh = plsc.VectorSubcoreMesh(core_axis_name="c", subcore_axis_name="s")

    @functools.partial(
        pl.kernel, mesh=mesh,
        out_type=[jax.ShapeDtypeStruct((e, dd), jnp.float32),
                  jax.ShapeDtypeStruct((e, dd), jnp.float32)],
        scratch_types=[
            pltpu.VMEM((nblk, _BE), jnp.int32),
            pltpu.VMEM((nblk, _BE), jnp.int32),
            pltpu.VMEM((_BE, dd), jnp.float32),
            pltpu.VMEM((_BE, dd), jnp.float32),
            pltpu.VMEM((_BE, dd), jnp.float32),
            pltpu.VMEM((_BE, dd), jnp.float32),
            pltpu.SemaphoreType.DMA,
            pltpu.SemaphoreType.DMA,
            pltpu.SemaphoreType.DMA,
            pltpu.SemaphoreType.DMA,
            pltpu.SemaphoreType.DMA,
            pltpu.SemaphoreType.DMA,
            pltpu.SemaphoreType.DMA,
            pltpu.SemaphoreType.DMA,
        ],
    )
    def k(s_hbm, fi_hbm, ti_hbm, sf_hbm, st_hbm,
          fiv, tiv, bf0, bt0, bf1, bt1,
          gf0, gt0, gf1, gt1, of0, ot0, of1, ot1):
        cid = lax.axis_index("c")
        sid = lax.axis_index("s")
        wid = sid * 2 + cid
        base = wid * (nblk * _BE)
        pltpu.sync_copy(fi_hbm.at[wid], fiv)
        pltpu.sync_copy(ti_hbm.at[wid], tiv)
        stages = ((bf0, bt0, gf0, gt0, of0, ot0),
                  (bf1, bt1, gf1, gt1, of1, ot1))

        def fire_g(j, bf, bt, gf, gt):
            pltpu.async_copy(s_hbm.at[fiv.at[j]], bf, gf)
            pltpu.async_copy(s_hbm.at[tiv.at[j]], bt, gt)

        fire_g(0, bf0, bt0, gf0, gt0)
        fire_g(1, bf1, bt1, gf1, gt1)

        def body(j, carry):
            for st in (0, 1):
                bf, bt, gf, gt, osf, ost = stages[st]

                @pl.when(lax.rem(j, 2) == st)
                def _():
                    pltpu.make_async_copy(s_hbm.at[fiv.at[j]], bf, gf).wait()
                    pltpu.make_async_copy(s_hbm.at[tiv.at[j]], bt, gt).wait()
                    r0 = base + j * _BE
                    co = pltpu.async_copy(bf, sf_hbm.at[pl.ds(r0, _BE)], osf)
                    ct = pltpu.async_copy(bt, st_hbm.at[pl.ds(r0, _BE)], ost)

                    @pl.when(j + 2 < nblk)
                    def _():
                        co.wait()
                        ct.wait()
                        fire_g(j + 2, bf, bt, gf, gt)
            return carry

        lax.fori_loop(0, nblk, body, 0)
        for st in (0, 1):
            bf, bt, gf, gt, osf, ost = stages[st]
            pltpu.make_async_copy(bf, sf_hbm.at[pl.ds(base, _BE)], osf).wait()
            pltpu.make_async_copy(bt, st_hbm.at[pl.ds(base, _BE)], ost).wait()

    return k(s_tab, fi3, ti3)


def _tail_body(ef_ref, sf_ref, st_ref, w1c_ref, b1_ref, w2_ref, b2_ref,
               w3_ref, b3_ref, e1_ref, e2_ref):
    d = ef_ref.shape[1]
    c = jnp.dot(ef_ref[...], w1c_ref[...],
                preferred_element_type=jnp.float32) + b1_ref[...]
    sf = sf_ref[...]
    st = st_ref[...]

    def head(g):
        h1 = jnp.maximum(g, 0.0)
        h2 = jnp.dot(h1, w2_ref[...], preferred_element_type=jnp.float32)
        h2 = jnp.maximum(h2 + b2_ref[...], 0.0)
        return jnp.dot(h2, w3_ref[...],
                       preferred_element_type=jnp.float32) + b3_ref[...]

    e1_ref[...] = head(sf[:, :d] + st[:, d:] + c)
    e2_ref[...] = head(st[:, :d] + sf[:, d:] + c)


def _mlp_tail(ef, sf, st, w1c, b1, w2, b2, w3, b3):
    e, d = ef.shape
    h = w1c.shape[1]
    be = 2000
    wspec = lambda shape: pl.BlockSpec(shape, lambda i: (0, 0))
    return pl.pallas_call(
        _tail_body,
        grid=(e // be,),
        in_specs=[pl.BlockSpec((be, d), lambda i: (i, 0)),
                  pl.BlockSpec((be, 2 * d), lambda i: (i, 0)),
                  pl.BlockSpec((be, 2 * d), lambda i: (i, 0)),
                  wspec(w1c.shape), wspec(b1.shape),
                  wspec(w2.shape), wspec(b2.shape),
                  wspec(w3.shape), wspec(b3.shape)],
        out_specs=[pl.BlockSpec((be, h), lambda i: (i, 0)),
                   pl.BlockSpec((be, h), lambda i: (i, 0))],
        out_shape=[jax.ShapeDtypeStruct((e, h), jnp.float32),
                   jax.ShapeDtypeStruct((e, h), jnp.float32)],
    )(ef, sf, st, w1c, b1, w2, b2, w3, b3)


def _sc_scatter(es1, es2, ti, fi, zeros):
    na, d = zeros.shape
    nblk = ti.shape[0] // (16 * _BE)
    rpt = na // 16
    wpr = _BE // 16
    mesh = plsc.VectorSubcoreMesh(core_axis_name="c", subcore_axis_name="s")

    @functools.partial(
        pl.kernel, mesh=mesh,
        out_type=jax.ShapeDtypeStruct((2 * _HN, d), jnp.float32),
        scratch_types=[
            pltpu.VMEM((_BE, d), jnp.float32),
            pltpu.VMEM((_BE, d), jnp.float32),
            pltpu.VMEM((_BE, d), jnp.float32),
            pltpu.VMEM((_BE, d), jnp.float32),
            pltpu.VMEM((_BE,), jnp.int32),
            pltpu.VMEM((_BE,), jnp.int32),
            pltpu.VMEM((_BE,), jnp.int32),
            pltpu.VMEM((_BE,), jnp.int32),
            pltpu.VMEM_SHARED((na, d), jnp.float32),
            pltpu.SemaphoreType.DMA,
            pltpu.SemaphoreType.DMA,
            pltpu.SemaphoreType.DMA,
            pltpu.SemaphoreType.DMA,
            pltpu.SemaphoreType.DMA,
            pltpu.SemaphoreType.DMA,
        ],
    )
    def k(es1_hbm, es2_hbm, ti_hbm, fi_hbm, z_hbm, out_hbm,
          b10, b20, b11, b21, it0, if0, it1, if1, acc,
          l0, l1, a10, a20, a11, a21):
        cid = lax.axis_index("c")
        sid = lax.axis_index("s")
        base = sid * (nblk * _BE)
        tr0 = sid * rpt
        pltpu.sync_copy(z_hbm.at[pl.ds(tr0, rpt)], acc.at[pl.ds(tr0, rpt)])
        plsc.subcore_barrier()
        lo = cid * _HN
        stages = ((b10, b20, it0, if0, l0, a10, a20),
                  (b11, b21, it1, if1, l1, a11, a21))

        def fire_l(j, b1, b2, it, if_, lsem):
            r0 = base + j * _BE
            pltpu.async_copy(es1_hbm.at[pl.ds(r0, _BE)], b1, lsem)
            pltpu.async_copy(es2_hbm.at[pl.ds(r0, _BE)], b2, lsem)
            pltpu.async_copy(ti_hbm.at[pl.ds(r0, _BE)], it, lsem)
            pltpu.async_copy(fi_hbm.at[pl.ds(r0, _BE)], if_, lsem)

        def wait_l(j, b1, b2, it, if_, lsem):
            r0 = base + j * _BE
            pltpu.make_async_copy(es1_hbm.at[pl.ds(r0, _BE)], b1, lsem).wait()
            pltpu.make_async_copy(es2_hbm.at[pl.ds(r0, _BE)], b2, lsem).wait()
            pltpu.make_async_copy(ti_hbm.at[pl.ds(r0, _BE)], it, lsem).wait()
            pltpu.make_async_copy(fi_hbm.at[pl.ds(r0, _BE)], if_, lsem).wait()

        def rebase(it, if_):
            for ref in (it, if_):
                for w in range(wpr):
                    v = ref[pl.ds(w * 16, 16)] - lo
                    oob = (v < 0) | (v >= _HN)
                    ref[pl.ds(w * 16, 16)] = jnp.where(oob, _HN, v)

        def stage_step(j, st_refs):
            b1, b2, it, if_, lsem, a1, a2 = st_refs
            wait_l(j, b1, b2, it, if_, lsem)
            rebase(it, if_)
            c1 = pltpu.async_copy(b1, acc.at[it], a1, add=True)
            c2 = pltpu.async_copy(b2, acc.at[if_], a2, add=True)
            return c1, c2

        fire_l(0, *stages[0][:5])
        fire_l(1, *stages[1][:5])

        def body(j, carry):
            for st in (0, 1):
                @pl.when(lax.rem(j, 2) == st)
                def _():
                    c1, c2 = stage_step(j, stages[st])
                    c1.wait()
                    c2.wait()
                    fire_l(j + 2, *stages[st][:5])
            return carry

        lax.fori_loop(0, nblk - 2, body, 0)
        for j in (nblk - 2, nblk - 1):
            c1, c2 = stage_step(j, stages[j % 2])
            c1.wait()
            c2.wait()
        plsc.subcore_barrier()
        drain = _HN // 16
        pltpu.sync_copy(acc.at[pl.ds(sid * drain, drain)],
                        out_hbm.at[pl.ds(cid * _HN + sid * drain, drain)])

    return k(es1, es2, ti, fi, zeros)


def _gru_body(ns_ref, agg_ref,
              wih0, whh0, bih0, bhh0,
              wih1, whh1, bih1, bhh1,
              wih2, whh2, bih2, bhh2, out_ref):
    d = ns_ref.shape[1]

    def cell(x, h, wih, whh, bih, bhh):
        gi = jnp.dot(x, wih[...], preferred_element_type=jnp.float32) + bih[...]
        gh = jnp.dot(h, whh[...], preferred_element_type=jnp.float32) + bhh[...]
        r = jax.nn.sigmoid(gi[:, :d] + gh[:, :d])
        z = jax.nn.sigmoid(gi[:, d:2 * d] + gh[:, d:2 * d])
        nn = jnp.tanh(gi[:, 2 * d:] + r * gh[:, 2 * d:])
        return (1.0 - z) * nn + z * h

    x0 = ns_ref[...]
    agg = agg_ref[...]
    n1 = cell(x0, agg, wih0, whh0, bih0, bhh0)
    n2 = cell(agg, n1, wih1, whh1, bih1, bhh1)
    out_ref[...] = cell(n1, n2, wih2, whh2, bih2, bhh2)


def _gru(ns, agg, weights):
    n, d = ns.shape
    bn = 2000
    wspecs = [pl.BlockSpec(w.shape, lambda i: (0, 0)) for w in weights]
    return pl.pallas_call(
        _gru_body,
        grid=(n // bn,),
        in_specs=[pl.BlockSpec((bn, d), lambda i: (i, 0)),
                  pl.BlockSpec((bn, d), lambda i: (i, 0))] + wspecs,
        out_specs=pl.BlockSpec((bn, d), lambda i: (i, 0)),
        out_shape=jax.ShapeDtypeStruct((n, d), jnp.float32),
    )(ns, agg, *weights)


def kernel(node_states, from_idx, to_idx, edge_features, graph_idx,
           mW1, mb1, mW2, mb2, mW3, mb3,
           g0_Wih, g0_Whh, g0_bih, g0_bhh,
           g1_Wih, g1_Whh, g1_bih, g1_bhh,
           g2_Wih, g2_Whh, g2_bih, g2_bhh):
    n, d = node_states.shape
    e = from_idx.shape[0]

    w_ab = jnp.concatenate([mW1[:d], mW1[d:2 * d]], axis=1)
    s_tab = _node_proj(node_states, w_ab)

    fi3 = from_idx.reshape(_NW, -1, _BE)
    ti3 = to_idx.reshape(_NW, -1, _BE)
    sf, st = _sc_gather(s_tab, fi3, ti3)

    es1, es2 = _mlp_tail(edge_features, sf, st,
                         mW1[2 * d:], mb1.reshape(1, -1),
                         mW2, mb2.reshape(1, -1),
                         mW3, mb3.reshape(1, -1))

    zeros = jnp.zeros((_NA, d), jnp.float32)
    agg = _sc_scatter(es1, es2, to_idx, from_idx, zeros)

    weights = [g0_Wih.T, g0_Whh.T, g0_bih.reshape(1, -1), g0_bhh.reshape(1, -1),
               g1_Wih.T, g1_Whh.T, g1_bih.reshape(1, -1), g1_bhh.reshape(1, -1),
               g2_Wih.T, g2_Whh.T, g2_bih.reshape(1, -1), g2_bhh.reshape(1, -1)]
    return _gru(node_states, agg[:n], weights)


# speedup vs baseline: 3.3724x; 1.2452x over previous
import functools

import jax
import jax.numpy as jnp
from jax import lax
from jax.experimental import pallas as pl
from jax.experimental.pallas import tpu as pltpu
from jax.experimental.pallas import tpu_sc as plsc

_NW = 32
_BE = 80
_BES = 40
_NA = 10240


def _proj_body(ns_ref, w_ref, s_ref):
    s_ref[...] = jnp.dot(ns_ref[...], w_ref[...],
                         preferred_element_type=jnp.float32)


def _node_proj(ns, w_ab):
    n, d = ns.shape
    bn = 2000
    return pl.pallas_call(
        _proj_body,
        grid=(n // bn,),
        in_specs=[pl.BlockSpec((bn, d), lambda i: (i, 0)),
                  pl.BlockSpec(w_ab.shape, lambda i: (0, 0))],
        out_specs=pl.BlockSpec((bn, w_ab.shape[1]), lambda i: (i, 0)),
        out_shape=jax.ShapeDtypeStruct((n, w_ab.shape[1]), jnp.float32),
    )(ns, w_ab)


def _sc_gather(s_tab, fi3, ti3):
    nblk = fi3.shape[1]
    e = _NW * nblk * _BE
    dd = s_tab.shape[1]
    mesh = plsc.VectorSubcoreMesh(core_axis_name="c", subcore_axis_name="s")

    @functools.partial(
        pl.kernel, mesh=mesh,
        out_type=[jax.ShapeDtypeStruct((e, dd), jnp.float32),
                  jax.ShapeDtypeStruct((e, dd), jnp.float32)],
        scratch_types=[
            pltpu.VMEM((nblk, _BE), jnp.int32),
            pltpu.VMEM((nblk, _BE), jnp.int32),
            pltpu.VMEM((_BE, dd), jnp.float32),
            pltpu.VMEM((_BE, dd), jnp.float32),
            pltpu.VMEM((_BE, dd), jnp.float32),
            pltpu.VMEM((_BE, dd), jnp.float32),
            pltpu.SemaphoreType.DMA,
            pltpu.SemaphoreType.DMA,
            pltpu.SemaphoreType.DMA,
            pltpu.SemaphoreType.DMA,
            pltpu.SemaphoreType.DMA,
            pltpu.SemaphoreType.DMA,
            pltpu.SemaphoreType.DMA,
            pltpu.SemaphoreType.DMA,
        ],
    )
    def k(s_hbm, fi_hbm, ti_hbm, sf_hbm, st_hbm,
          fiv, tiv, bf0, bt0, bf1, bt1,
          gf0, gt0, gf1, gt1, of0, ot0, of1, ot1):
        cid = lax.axis_index("c")
        sid = lax.axis_index("s")
        wid = sid * 2 + cid
        base = wid * (nblk * _BE)
        pltpu.sync_copy(fi_hbm.at[wid], fiv)
        pltpu.sync_copy(ti_hbm.at[wid], tiv)
        stages = ((bf0, bt0, gf0, gt0, of0, ot0),
                  (bf1, bt1, gf1, gt1, of1, ot1))

        def fire_g(j, bf, bt, gf, gt):
            pltpu.async_copy(s_hbm.at[fiv.at[j]], bf, gf)
            pltpu.async_copy(s_hbm.at[tiv.at[j]], bt, gt)

        fire_g(0, bf0, bt0, gf0, gt0)
        fire_g(1, bf1, bt1, gf1, gt1)

        def body(j, carry):
            for st in (0, 1):
                bf, bt, gf, gt, osf, ost = stages[st]

                @pl.when(lax.rem(j, 2) == st)
                def _():
                    pltpu.make_async_copy(s_hbm.at[fiv.at[j]], bf, gf).wait()
                    pltpu.make_async_copy(s_hbm.at[tiv.at[j]], bt, gt).wait()
                    r0 = base + j * _BE
                    co = pltpu.async_copy(bf, sf_hbm.at[pl.ds(r0, _BE)], osf)
                    ct = pltpu.async_copy(bt, st_hbm.at[pl.ds(r0, _BE)], ost)

                    @pl.when(j + 2 < nblk)
                    def _():
                        co.wait()
                        ct.wait()
                        fire_g(j + 2, bf, bt, gf, gt)
            return carry

        lax.fori_loop(0, nblk, body, 0)
        for st in (0, 1):
            bf, bt, gf, gt, osf, ost = stages[st]
            pltpu.make_async_copy(bf, sf_hbm.at[pl.ds(base, _BE)], osf).wait()
            pltpu.make_async_copy(bt, st_hbm.at[pl.ds(base, _BE)], ost).wait()

    return k(s_tab, fi3, ti3)


def _tail_body(ef_ref, sf_ref, st_ref, w1c_ref, b1_ref, w2_ref, b2_ref,
               w3_ref, b3_ref, e1_ref, e2_ref):
    d = ef_ref.shape[1]
    c = jnp.dot(ef_ref[...], w1c_ref[...],
                preferred_element_type=jnp.float32) + b1_ref[...]
    sf = sf_ref[...]
    st = st_ref[...]

    def head(g):
        h1 = jnp.maximum(g, 0.0)
        h2 = jnp.dot(h1, w2_ref[...], preferred_element_type=jnp.float32)
        h2 = jnp.maximum(h2 + b2_ref[...], 0.0)
        return jnp.dot(h2, w3_ref[...],
                       preferred_element_type=jnp.float32) + b3_ref[...]

    e1_ref[...] = head(sf[:, :d] + st[:, d:] + c)
    e2_ref[...] = head(st[:, :d] + sf[:, d:] + c)


def _mlp_tail(ef, sf, st, w1c, b1, w2, b2, w3, b3):
    e, d = ef.shape
    h = w1c.shape[1]
    be = 2000
    wspec = lambda shape: pl.BlockSpec(shape, lambda i: (0, 0))
    return pl.pallas_call(
        _tail_body,
        grid=(e // be,),
        in_specs=[pl.BlockSpec((be, d), lambda i: (i, 0)),
                  pl.BlockSpec((be, 2 * d), lambda i: (i, 0)),
                  pl.BlockSpec((be, 2 * d), lambda i: (i, 0)),
                  wspec(w1c.shape), wspec(b1.shape),
                  wspec(w2.shape), wspec(b2.shape),
                  wspec(w3.shape), wspec(b3.shape)],
        out_specs=[pl.BlockSpec((be, h), lambda i: (i, 0)),
                   pl.BlockSpec((be, h), lambda i: (i, 0))],
        out_shape=[jax.ShapeDtypeStruct((e, h), jnp.float32),
                   jax.ShapeDtypeStruct((e, h), jnp.float32)],
    )(ef, sf, st, w1c, b1, w2, b2, w3, b3)


def _sc_scatter(es1, es2, ti, fi, zeros):
    na, d = zeros.shape
    nblk = ti.shape[0] // (_NW * _BES)
    rpt = na // 16
    mesh = plsc.VectorSubcoreMesh(core_axis_name="c", subcore_axis_name="s")

    @functools.partial(
        pl.kernel, mesh=mesh,
        out_type=jax.ShapeDtypeStruct((2 * na, d), jnp.float32),
        scratch_types=[
            pltpu.VMEM((_BES, d), jnp.float32),
            pltpu.VMEM((_BES, d), jnp.float32),
            pltpu.VMEM((_BES, d), jnp.float32),
            pltpu.VMEM((_BES, d), jnp.float32),
            pltpu.VMEM((_BES,), jnp.int32),
            pltpu.VMEM((_BES,), jnp.int32),
            pltpu.VMEM((_BES,), jnp.int32),
            pltpu.VMEM((_BES,), jnp.int32),
            pltpu.VMEM_SHARED((na, d), jnp.float32),
            pltpu.SemaphoreType.DMA,
            pltpu.SemaphoreType.DMA,
            pltpu.SemaphoreType.DMA,
            pltpu.SemaphoreType.DMA,
            pltpu.SemaphoreType.DMA,
            pltpu.SemaphoreType.DMA,
        ],
    )
    def k(es1_hbm, es2_hbm, ti_hbm, fi_hbm, z_hbm, out_hbm,
          b10, b20, b11, b21, it0, if0, it1, if1, acc,
          l0, l1, a10, a20, a11, a21):
        cid = lax.axis_index("c")
        sid = lax.axis_index("s")
        wid = sid * 2 + cid
        base = wid * (nblk * _BES)
        tr0 = sid * rpt
        pltpu.sync_copy(z_hbm.at[pl.ds(tr0, rpt)], acc.at[pl.ds(tr0, rpt)])
        plsc.subcore_barrier()
        stages = ((b10, b20, it0, if0, l0, a10, a20),
                  (b11, b21, it1, if1, l1, a11, a21))

        def fire_l(j, b1, b2, it, if_, lsem):
            r0 = base + j * _BES
            pltpu.async_copy(es1_hbm.at[pl.ds(r0, _BES)], b1, lsem)
            pltpu.async_copy(es2_hbm.at[pl.ds(r0, _BES)], b2, lsem)
            pltpu.async_copy(ti_hbm.at[pl.ds(r0, _BES)], it, lsem)
            pltpu.async_copy(fi_hbm.at[pl.ds(r0, _BES)], if_, lsem)

        def wait_l(j, b1, b2, it, if_, lsem):
            r0 = base + j * _BES
            pltpu.make_async_copy(es1_hbm.at[pl.ds(r0, _BES)], b1, lsem).wait()
            pltpu.make_async_copy(es2_hbm.at[pl.ds(r0, _BES)], b2, lsem).wait()
            pltpu.make_async_copy(ti_hbm.at[pl.ds(r0, _BES)], it, lsem).wait()
            pltpu.make_async_copy(fi_hbm.at[pl.ds(r0, _BES)], if_, lsem).wait()

        def stage_step(j, st_refs):
            b1, b2, it, if_, lsem, a1, a2 = st_refs
            wait_l(j, b1, b2, it, if_, lsem)
            c1 = pltpu.async_copy(b1, acc.at[it], a1, add=True)
            c2 = pltpu.async_copy(b2, acc.at[if_], a2, add=True)
            return c1, c2

        fire_l(0, *stages[0][:5])
        fire_l(1, *stages[1][:5])

        def body(j, carry):
            for st in (0, 1):
                @pl.when(lax.rem(j, 2) == st)
                def _():
                    c1, c2 = stage_step(j, stages[st])
                    c1.wait()
                    c2.wait()
                    fire_l(j + 2, *stages[st][:5])
            return carry

        lax.fori_loop(0, nblk - 2, body, 0)
        for j in (nblk - 2, nblk - 1):
            c1, c2 = stage_step(j, stages[j % 2])
            c1.wait()
            c2.wait()
        plsc.subcore_barrier()
        pltpu.sync_copy(acc.at[pl.ds(tr0, rpt)],
                        out_hbm.at[pl.ds(cid * na + tr0, rpt)])

    return k(es1, es2, ti, fi, zeros)


def _gru_body(ns_ref, p0_ref, p1_ref,
              wih0, whh0, bih0, bhh0,
              wih1, whh1, bih1, bhh1,
              wih2, whh2, bih2, bhh2, out_ref):
    d = ns_ref.shape[1]

    def cell(x, h, wih, whh, bih, bhh):
        gi = jnp.dot(x, wih[...], preferred_element_type=jnp.float32) + bih[...]
        gh = jnp.dot(h, whh[...], preferred_element_type=jnp.float32) + bhh[...]
        r = jax.nn.sigmoid(gi[:, :d] + gh[:, :d])
        z = jax.nn.sigmoid(gi[:, d:2 * d] + gh[:, d:2 * d])
        nn = jnp.tanh(gi[:, 2 * d:] + r * gh[:, 2 * d:])
        return (1.0 - z) * nn + z * h

    x0 = ns_ref[...]
    agg = p0_ref[...] + p1_ref[...]
    n1 = cell(x0, agg, wih0, whh0, bih0, bhh0)
    n2 = cell(agg, n1, wih1, whh1, bih1, bhh1)
    out_ref[...] = cell(n1, n2, wih2, whh2, bih2, bhh2)


def _gru(ns, p0, p1, weights):
    n, d = ns.shape
    bn = 2000
    wspecs = [pl.BlockSpec(w.shape, lambda i: (0, 0)) for w in weights]
    return pl.pallas_call(
        _gru_body,
        grid=(n // bn,),
        in_specs=[pl.BlockSpec((bn, d), lambda i: (i, 0)),
                  pl.BlockSpec((bn, d), lambda i: (i, 0)),
                  pl.BlockSpec((bn, d), lambda i: (i, 0))] + wspecs,
        out_specs=pl.BlockSpec((bn, d), lambda i: (i, 0)),
        out_shape=jax.ShapeDtypeStruct((n, d), jnp.float32),
    )(ns, p0, p1, *weights)


def kernel(node_states, from_idx, to_idx, edge_features, graph_idx,
           mW1, mb1, mW2, mb2, mW3, mb3,
           g0_Wih, g0_Whh, g0_bih, g0_bhh,
           g1_Wih, g1_Whh, g1_bih, g1_bhh,
           g2_Wih, g2_Whh, g2_bih, g2_bhh):
    n, d = node_states.shape
    e = from_idx.shape[0]

    w_ab = jnp.concatenate([mW1[:d], mW1[d:2 * d]], axis=1)
    s_tab = _node_proj(node_states, w_ab)

    fi3 = from_idx.reshape(_NW, -1, _BE)
    ti3 = to_idx.reshape(_NW, -1, _BE)
    sf, st = _sc_gather(s_tab, fi3, ti3)

    es1, es2 = _mlp_tail(edge_features, sf, st,
                         mW1[2 * d:], mb1.reshape(1, -1),
                         mW2, mb2.reshape(1, -1),
                         mW3, mb3.reshape(1, -1))

    zeros = jnp.zeros((_NA, d), jnp.float32)
    parts = _sc_scatter(es1, es2, to_idx, from_idx, zeros)

    weights = [g0_Wih.T, g0_Whh.T, g0_bih.reshape(1, -1), g0_bhh.reshape(1, -1),
               g1_Wih.T, g1_Whh.T, g1_bih.reshape(1, -1), g1_bhh.reshape(1, -1),
               g2_Wih.T, g2_Whh.T, g2_bih.reshape(1, -1), g2_bhh.reshape(1, -1)]
    return _gru(node_states, parts[:n], parts[_NA:_NA + n], weights)
